# baseline (speedup 1.0000x reference)
import functools
import jax
import jax.numpy as jnp
from jax import lax
from jax.experimental import pallas as pl
from jax.experimental.pallas import tpu as pltpu
from jax.experimental.pallas import tpu_sc as plsc

N = 10000
E = 320000
IN = 128
H = 128
OUT = 64
HEADS = 8
DH = 16

NC = 2
NS = 16
NW = NC * NS
B = 64
EPW = 10112
NB = EPW // B
EPAD = EPW * NW
NPAD = 10016
RPT = NPAD // NS
ZB = 128
RCH = RPT // ZB
REM = RPT % ZB

NBLK = 1000
NNB = N // NBLK

_MESH = dict(core_axis_name="c", subcore_axis_name="s", num_cores=NC,
             num_subcores=NS)
_SC_PARAMS = pltpu.CompilerParams(needs_layout_passes=False,
                                  use_tc_tiling_on_sc=False)
_SC_PARAMS_TILED = _SC_PARAMS

F32 = jnp.float32
I32 = jnp.int32


def _leaky(t):
    return jnp.where(t > 0, t, 0.2 * t)


def _rowspec(d):
    return pl.BlockSpec((NBLK, d), lambda k: (k, 0))


def _fullspec(shape):
    nd = len(shape)
    return pl.BlockSpec(shape, lambda k, _n=nd: (0,) * _n)


def _partspec(d):
    return pl.BlockSpec((2, NBLK, d), lambda k: (0, k, 0))


def _affine(stats_row, g, beta):
    mu = stats_row[0] / N
    var = stats_row[1] / N - mu * mu
    s = g * lax.rsqrt(var + 1e-5)
    return s, beta - mu * s


def _acc_stats(acc_ref, out_ref, y, k):
    @pl.when(k == 0)
    def _():
        acc_ref[...] = jnp.zeros_like(acc_ref)

    acc_ref[...] += jnp.concatenate(
        [jnp.sum(y, axis=0)[None], jnp.sum(y * y, axis=0)[None]], axis=0)

    @pl.when(k == NNB - 1)
    def _():
        out_ref[...] = acc_ref[...]


def _rowspec2(d):
    return pl.BlockSpec((NBLK, d), lambda k: (k % NNB, 0))


def _acc_stats2(acc_ref, y, k):
    @pl.when(k == 0)
    def _():
        acc_ref[...] = jnp.zeros_like(acc_ref)

    acc_ref[...] += jnp.concatenate(
        [jnp.sum(y, axis=0)[None], jnp.sum(y * y, axis=0)[None]], axis=0)


def _k1_body(x_ref, w_ref, b_ref, g_ref, beta_ref, gw_ref, wa_ref,
             h0_ref, hh_ref, asd_ref, mx_ref, ybuf, acc_ref, mxa_ref):
    k = pl.program_id(0)

    @pl.when(k < NNB)
    def _():
        y = x_ref[...] @ w_ref[...] + b_ref[...]
        ybuf[pl.ds(k * NBLK, NBLK), :] = y
        _acc_stats2(acc_ref, y, k)

    @pl.when(k >= NNB)
    def _():
        s, c = _affine(acc_ref[...], g_ref[...], beta_ref[...])
        y = ybuf[pl.ds((k - NNB) * NBLK, NBLK), :]
        h0 = jax.nn.relu(y * s + c)
        hh = h0 @ gw_ref[...]
        asd = hh @ wa_ref[...]
        h0_ref[...] = h0
        hh_ref[...] = hh
        asd_ref[...] = asd

        @pl.when(k == NNB)
        def _():
            mxa_ref[...] = jnp.full_like(mxa_ref, -jnp.inf)

        mxa_ref[...] = jnp.maximum(mxa_ref[...],
                                   jnp.max(asd, axis=0)[None])

        @pl.when(k == 2 * NNB - 1)
        def _():
            mx = mxa_ref[...]
            swapped = jnp.concatenate([mx[:, HEADS:], mx[:, :HEADS]], axis=1)
            mx_ref[...] = jnp.concatenate([mx, swapped], axis=0)


def _k1(x, wa, p):
    return pl.pallas_call(
        _k1_body,
        grid=(2 * NNB,),
        in_specs=[_rowspec2(IN), _fullspec((IN, H)), _fullspec((H,)),
                  _fullspec((H,)), _fullspec((H,)), _fullspec((H, H)),
                  _fullspec((H, 2 * HEADS))],
        out_specs=(_rowspec2(H), _rowspec2(H), _rowspec2(2 * HEADS),
                   _fullspec((2, 2 * HEADS))),
        out_shape=(jax.ShapeDtypeStruct((N, H), F32),
                   jax.ShapeDtypeStruct((N, H), F32),
                   jax.ShapeDtypeStruct((N, 2 * HEADS), F32),
                   jax.ShapeDtypeStruct((2, 2 * HEADS), F32)),
        scratch_shapes=[pltpu.VMEM((N, H), F32),
                        pltpu.VMEM((2, H), F32),
                        pltpu.VMEM((1, 2 * HEADS), F32)],
    )(x, p['in_W'], p['in_b'], p['in_g'], p['in_beta'], p['gat_W'], wa)


def _partspec2(d):
    return pl.BlockSpec((2, NBLK, d), lambda k: (0, k % NNB, 0))


def _k2_body(denp_ref, nump_ref, asd_ref, mx_ref, hh_ref, gb_ref, g_ref,
             beta_ref, h0_ref, h1_ref, ybuf, acc_ref):
    k = pl.program_id(0)

    @pl.when(k < NNB)
    def _():
        a_s = asd_ref[...][:, :HEADS]
        a_d = asd_ref[...][:, HEADS:]
        mx = mx_ref[...]
        m = _leaky(mx[0:1, :HEADS] + mx[0:1, HEADS:])
        vs = jnp.exp(_leaky(a_s + a_d) - m)
        den = (denp_ref[...][0, :, :HEADS] + denp_ref[...][1, :, :HEADS]
               + vs + 1e-16)
        hh = hh_ref[...]
        vrep = jnp.repeat(vs, DH, axis=1)
        drep = jnp.repeat(den, DH, axis=1)
        num = nump_ref[...][0] + nump_ref[...][1] + hh * vrep
        gat = num / drep + gb_ref[...]
        ybuf[pl.ds(k * NBLK, NBLK), :] = gat
        _acc_stats2(acc_ref, gat, k)

    @pl.when(k >= NNB)
    def _():
        s, c = _affine(acc_ref[...], g_ref[...], beta_ref[...])
        y = ybuf[pl.ds((k - NNB) * NBLK, NBLK), :]
        h1_ref[...] = jax.nn.relu(y * s + c) + h0_ref[...]


def _k2(den_p, num_p, asd, mx, hh, h0, p):
    return pl.pallas_call(
        _k2_body,
        grid=(2 * NNB,),
        in_specs=[_partspec2(2 * HEADS), _partspec2(H), _rowspec2(2 * HEADS),
                  _fullspec((2, 2 * HEADS)), _rowspec2(H), _fullspec((H,)),
                  _fullspec((H,)), _fullspec((H,)), _rowspec2(H)],
        out_specs=_rowspec2(H),
        out_shape=jax.ShapeDtypeStruct((N, H), F32),
        scratch_shapes=[pltpu.VMEM((N, H), F32), pltpu.VMEM((2, H), F32)],
    )(den_p, num_p, asd, mx, hh, p['gat_b'], p['bn0_g'], p['bn0_b'], h0)


def _k3_body(aggp_ref, denp_ref, h_ref, wl_ref, bl_ref, wr_ref, g_ref,
             beta_ref, o_ref, ybuf, acc_ref, *, resid):
    k = pl.program_id(0)

    @pl.when(k < NNB)
    def _():
        deg = (denp_ref[...][0, :, HEADS:HEADS + 1]
               + denp_ref[...][1, :, HEADS:HEADS + 1])
        inv = 1.0 / jnp.maximum(deg, 1.0)
        agg = (aggp_ref[...][0] + aggp_ref[...][1]) * inv
        y = agg @ wl_ref[...] + bl_ref[...] + h_ref[...] @ wr_ref[...]
        ybuf[pl.ds(k * NBLK, NBLK), :] = y
        _acc_stats2(acc_ref, y, k)

    @pl.when(k >= NNB)
    def _():
        s, c = _affine(acc_ref[...], g_ref[...], beta_ref[...])
        y = ybuf[pl.ds((k - NNB) * NBLK, NBLK), :]
        o = y * s + c
        if resid:
            o = jax.nn.relu(o) + h_ref[...]
        o_ref[...] = o


def _k3(agg_p, den_p, h, wl, bl, wr, g, beta, dout, resid):
    return pl.pallas_call(
        functools.partial(_k3_body, resid=resid),
        grid=(2 * NNB,),
        in_specs=[_partspec2(H), _partspec2(2 * HEADS), _rowspec2(H),
                  _fullspec((H, dout)), _fullspec((dout,)),
                  _fullspec((H, dout)), _fullspec((dout,)),
                  _fullspec((dout,))],
        out_specs=_rowspec2(dout),
        out_shape=jax.ShapeDtypeStruct((N, dout), F32),
        scratch_shapes=[pltpu.VMEM((N, dout), F32), pltpu.VMEM((2, dout), F32)],
    )(agg_p, den_p, h, wl, bl, wr, g, beta)


def _k4a_body(node_ref, w1u_ref, w1v_ref, a1_ref, a2_ref):
    node = node_ref[...]
    a1_ref[...] = node @ w1u_ref[...]
    a2_ref[...] = node @ w1v_ref[...]


def _k4a(node, p):
    return pl.pallas_call(
        _k4a_body,
        grid=(NNB,),
        in_specs=[_rowspec(OUT), _fullspec((OUT, H)), _fullspec((OUT, H))],
        out_specs=(_rowspec(H), _rowspec(H)),
        out_shape=(jax.ShapeDtypeStruct((N, H), F32),
                   jax.ShapeDtypeStruct((N, H), F32)),
    )(node, p['cf_W1'][:OUT], p['cf_W1'][OUT:2 * OUT])


def _k4h_body(node_ref, sw1_ref, sb1_ref, lw_ref, lb_ref, pw_ref, pb_ref,
              t_ref, loc_ref, perf_ref, s_ref, acc_ref):
    node = node_ref[...]
    t = node @ sw1_ref[...] + sb1_ref[...]
    t_ref[...] = t
    loc_ref[...] = node @ lw_ref[...] + lb_ref[...]
    perf_ref[...] = node @ pw_ref[...] + pb_ref[...]
    _acc_stats(acc_ref, s_ref, t, pl.program_id(0))


def _k4h(node, p):
    return pl.pallas_call(
        _k4h_body,
        grid=(NNB,),
        in_specs=[_rowspec(OUT), _fullspec((OUT, H)), _fullspec((H,)),
                  _fullspec((OUT, 3)), _fullspec((3,)), _fullspec((OUT, 1)),
                  _fullspec((1,))],
        out_specs=(_rowspec(H), _rowspec(3), _rowspec(1), _fullspec((2, H))),
        out_shape=(jax.ShapeDtypeStruct((N, H), F32),
                   jax.ShapeDtypeStruct((N, 3), F32),
                   jax.ShapeDtypeStruct((N, 1), F32),
                   jax.ShapeDtypeStruct((2, H), F32)),
        scratch_shapes=[pltpu.VMEM((2, H), F32)],
    )(node, p['sc_W1'], p['sc_b1'], p['loc_W'], p['loc_b'], p['perf_W'],
      p['perf_b'])


def _k4b_body(t_ref, s_ref, g_ref, beta_ref, w2_ref, b2_ref, w3_ref, b3_ref,
              sup_ref):
    s, c = _affine(s_ref[...], g_ref[...], beta_ref[...])
    z = jax.nn.relu(t_ref[...] * s + c)
    z = jax.nn.relu(z @ w2_ref[...] + b2_ref[...])
    sup_ref[...] = z @ w3_ref[...] + b3_ref[...]


def _k4b(t, stats, p):
    return pl.pallas_call(
        _k4b_body,
        grid=(NNB,),
        in_specs=[_rowspec(H), _fullspec((2, H)), _fullspec((H,)),
                  _fullspec((H,)), _fullspec((H, OUT)), _fullspec((OUT,)),
                  _fullspec((OUT, 4)), _fullspec((4,))],
        out_specs=_rowspec(4),
        out_shape=jax.ShapeDtypeStruct((N, 4), F32),
    )(t, stats, p['sc_g'], p['sc_beta'], p['sc_W2'], p['sc_b2'], p['sc_W3'],
      p['sc_b3'])


_YBLK = 512
_NYB = E // _YBLK


def _stats_body(y_ref, s_ref, acc_ref):
    k = pl.program_id(0)

    @pl.when(k == 0)
    def _():
        acc_ref[...] = jnp.zeros_like(acc_ref)

    y = y_ref[...]
    acc_ref[...] += jnp.concatenate(
        [jnp.sum(y, axis=0)[None], jnp.sum(y * y, axis=0)[None]], axis=0)

    @pl.when(k == _NYB - 1)
    def _():
        s_ref[...] = acc_ref[...]


def _tc_stats(y):
    return pl.pallas_call(
        _stats_body,
        grid=(_NYB,),
        in_specs=[pl.BlockSpec((_YBLK, H), lambda k: (k, 0))],
        out_specs=pl.BlockSpec((2, H), lambda k: (0, 0)),
        out_shape=jax.ShapeDtypeStruct((2, H), F32),
        scratch_shapes=[pltpu.VMEM((2, H), F32)],
    )(y)


def _carbon_body(y_ref, s_ref, g_ref, beta_ref, w2_ref, b2_ref, w3_ref,
                 b3_ref, out_ref):
    mu = s_ref[...][0] / E
    var = s_ref[...][1] / E - mu * mu
    sc = g_ref[...] * lax.rsqrt(var + 1e-5)
    cc = beta_ref[...] - mu * sc
    z = jax.nn.relu(y_ref[...] * sc + cc)
    z = jax.nn.relu(z @ w2_ref[...] + b2_ref[...])
    out_ref[...] = z @ w3_ref[...] + b3_ref[...]


def _tc_carbon(y, stats, p):
    return pl.pallas_call(
        _carbon_body,
        grid=(_NYB,),
        in_specs=[
            pl.BlockSpec((_YBLK, H), lambda k: (k, 0)),
            pl.BlockSpec((2, H), lambda k: (0, 0)),
            pl.BlockSpec((H,), lambda k: (0,)),
            pl.BlockSpec((H,), lambda k: (0,)),
            pl.BlockSpec((H, OUT), lambda k: (0, 0)),
            pl.BlockSpec((OUT,), lambda k: (0,)),
            pl.BlockSpec((OUT, 1), lambda k: (0, 0)),
            pl.BlockSpec((1,), lambda k: (0,)),
        ],
        out_specs=pl.BlockSpec((_YBLK, 1), lambda k: (k, 0)),
        out_shape=jax.ShapeDtypeStruct((E, 1), F32),
    )(y, stats, p['cf_g'], p['cf_beta'], p['cf_W2'], p['cf_b2'], p['cf_W3'],
      p['cf_b3'])


def _zero_rows(buf, rows):
    cols = buf.shape[1]
    zero = jnp.zeros((16,), F32)

    @pl.loop(0, rows)
    def _(r):
        @pl.loop(0, cols, step=16)
        def _(c0):
            buf[r, pl.ds(c0, 16)] = zero


def _zero_spmem(zb, dst_s, r0):
    rows = zb.shape[0]
    n_full = RPT // rows
    rem = RPT % rows
    for j in range(n_full):
        pltpu.sync_copy(zb, dst_s.at[pl.ds(r0 + j * rows, rows)])
    if rem:
        pltpu.sync_copy(zb.at[pl.ds(0, rem)],
                        dst_s.at[pl.ds(r0 + n_full * rows, rem)])


def _unpack_idx(pk, ixs_b, ixd_b, k, s):
    @pl.loop(0, B, step=16)
    def _(c):
        p = pk[k, pl.ds(c, 16)]
        ixs_b[s, pl.ds(c, 16)] = p & 0x3FFF
        ixd_b[s, pl.ds(c, 16)] = lax.shift_right_logical(p, 14)


def _sc_gat_body(ts_hbm, td_hbm, hh_hbm, pk_hbm, m_hbm,
                 den_hbm, num_hbm,
                 pk, ixs, ixd, bs, bd, bh, vb, nb_, mbuf, den_s, num_s, gsem,
                 esem):
    cid = lax.axis_index("c")
    sid = lax.axis_index("s")
    wid = sid * NC + cid
    r0 = sid * RPT

    _zero_rows(nb_, B)
    _zero_rows(vb, B)
    _zero_spmem(nb_, num_s, r0)
    _zero_spmem(vb, den_s, r0)
    pltpu.sync_copy(m_hbm, mbuf)
    pltpu.sync_copy(pk_hbm.at[pl.ds(wid * NB, NB)], pk)
    plsc.subcore_barrier()

    lane = lax.iota(I32, 16)
    is_head = lane < HEADS
    is_deg = lane == HEADS
    mr = mbuf[0] + mbuf[1]
    mvec = jnp.maximum(mr, 0.2 * mr)

    def issue(k, s):
        _unpack_idx(pk, ixs, ixd, k, s)
        pltpu.make_async_copy(ts_hbm.at[ixs.at[s]], bs.at[s],
                              gsem.at[s]).start()
        pltpu.make_async_copy(td_hbm.at[ixd.at[s]], bd.at[s],
                              gsem.at[s]).start()
        pltpu.make_async_copy(hh_hbm.at[ixs.at[s]], bh.at[s],
                              gsem.at[s]).start()

    def drain(s):
        pltpu.make_async_copy(ts_hbm.at[ixs.at[s]], bs.at[s],
                              gsem.at[s]).wait()
        pltpu.make_async_copy(td_hbm.at[ixd.at[s]], bd.at[s],
                              gsem.at[s]).wait()
        pltpu.make_async_copy(hh_hbm.at[ixs.at[s]], bh.at[s],
                              gsem.at[s]).wait()

    def compute(s):
        @pl.loop(0, B, unroll=4)
        def _(i):
            t = bs[s, i] + bd[s, i]
            v = jnp.exp(jnp.maximum(t, 0.2 * t) - mvec)
            v = jnp.where(is_head, v,
                          jnp.where(is_deg, jnp.ones((16,), F32),
                                    jnp.zeros((16,), F32)))
            vb[i] = v
            for h in range(HEADS):
                vh = plsc.load_gather(
                    vb, [jnp.full((16,), i, I32), jnp.full((16,), h, I32)])
                nb_[i, pl.ds(h * DH, DH)] = bh[s, i, pl.ds(h * DH, DH)] * vh

        pltpu.sync_copy(vb, den_s.at[ixd.at[s]], add=True)
        pltpu.sync_copy(nb_, num_s.at[ixd.at[s]], add=True)

    issue(0, 0)

    @pl.loop(0, NB, step=2)
    def _(k):
        issue(k + 1, 1)
        drain(0)
        compute(0)

        @pl.when(k + 2 < NB)
        def _():
            issue(k + 2, 0)

        drain(1)
        compute(1)

    plsc.subcore_barrier()
    pltpu.async_copy(den_s.at[pl.ds(r0, RPT)],
                     den_hbm.at[cid, pl.ds(r0, RPT)], esem).wait()
    pltpu.async_copy(num_s.at[pl.ds(r0, RPT)],
                     num_hbm.at[cid, pl.ds(r0, RPT)], esem).wait()


def _sc_gat(ts, td, hh, pk2, m):
    f = pl.kernel(
        _sc_gat_body,
        out_type=(
            jax.ShapeDtypeStruct((NC, NPAD, 16), F32),
            jax.ShapeDtypeStruct((NC, NPAD, H), F32),
        ),
        mesh=plsc.VectorSubcoreMesh(**_MESH),
        compiler_params=_SC_PARAMS,
        scratch_types=[
            pltpu.VMEM((NB, B), I32),
            pltpu.VMEM((2, B), I32),
            pltpu.VMEM((2, B), I32),
            pltpu.VMEM((2, B, 16), F32),
            pltpu.VMEM((2, B, 16), F32),
            pltpu.VMEM((2, B, H), F32),
            pltpu.VMEM((B, 16), F32),
            pltpu.VMEM((B, H), F32),
            pltpu.VMEM((2, 16), F32),
            pltpu.VMEM_SHARED((NPAD, 16), F32),
            pltpu.VMEM_SHARED((NPAD, H), F32),
            pltpu.SemaphoreType.DMA((2,)),
            pltpu.SemaphoreType.DMA,
        ],
    )
    return f(ts, td, hh, pk2, m)


def _sc_agg_body(h_hbm, pk_hbm, agg_hbm, pk, ixs, ixd, bh, acc_s,
                 gsem, esem):
    cid = lax.axis_index("c")
    sid = lax.axis_index("s")
    wid = sid * NC + cid
    r0 = sid * RPT

    _zero_rows(bh.at[0], B)
    _zero_spmem(bh.at[0], acc_s, r0)
    pltpu.sync_copy(pk_hbm.at[pl.ds(wid * NB, NB)], pk)
    plsc.subcore_barrier()

    def issue(k, s):
        _unpack_idx(pk, ixs, ixd, k, s)
        pltpu.make_async_copy(h_hbm.at[ixs.at[s]], bh.at[s],
                              gsem.at[s]).start()

    def flush(s):
        pltpu.make_async_copy(h_hbm.at[ixs.at[s]], bh.at[s],
                              gsem.at[s]).wait()
        pltpu.sync_copy(bh.at[s], acc_s.at[ixd.at[s]], add=True)

    issue(0, 0)

    @pl.loop(0, NB, step=2)
    def _(k):
        issue(k + 1, 1)
        flush(0)

        @pl.when(k + 2 < NB)
        def _():
            issue(k + 2, 0)

        flush(1)

    plsc.subcore_barrier()
    pltpu.async_copy(acc_s.at[pl.ds(r0, RPT)],
                     agg_hbm.at[cid, pl.ds(r0, RPT)], esem).wait()


def _sc_agg(h, pk2):
    f = pl.kernel(
        _sc_agg_body,
        out_type=jax.ShapeDtypeStruct((NC, NPAD, H), F32),
        mesh=plsc.VectorSubcoreMesh(**_MESH),
        compiler_params=_SC_PARAMS_TILED,
        scratch_types=[
            pltpu.VMEM((NB, B), I32),
            pltpu.VMEM((2, B), I32),
            pltpu.VMEM((2, B), I32),
            pltpu.VMEM((2, B, H), F32),
            pltpu.VMEM_SHARED((NPAD, H), F32),
            pltpu.SemaphoreType.DMA((2,)),
            pltpu.SemaphoreType.DMA,
        ],
    )
    return f(h, pk2)


_EAW = 3 * EPW


def _sc_edge_body(a1_hbm, a2_hbm, ea_hbm, pk_hbm, w1a_hbm, y_hbm,
                  pk, ixs, ixd, b1, b2, ba, yb, wbuf, gsem, ysem):
    cid = lax.axis_index("c")
    sid = lax.axis_index("s")
    wid = sid * NC + cid

    pltpu.sync_copy(w1a_hbm, wbuf)
    pltpu.sync_copy(pk_hbm.at[pl.ds(wid * NB, NB)], pk)
    pltpu.sync_copy(ea_hbm.at[pl.ds(wid * _EAW, _EAW)],
                    ba.at[pl.ds(0, _EAW)])

    def issue(k, s):
        _unpack_idx(pk, ixs, ixd, k, s)
        pltpu.make_async_copy(a1_hbm.at[ixs.at[s]], b1.at[s],
                              gsem.at[s]).start()
        pltpu.make_async_copy(a2_hbm.at[ixd.at[s]], b2.at[s],
                              gsem.at[s]).start()

    def drain(s):
        pltpu.make_async_copy(a1_hbm.at[ixs.at[s]], b1.at[s],
                              gsem.at[s]).wait()
        pltpu.make_async_copy(a2_hbm.at[ixd.at[s]], b2.at[s],
                              gsem.at[s]).wait()

    def compute(k, s):
        @pl.when(k >= 2)
        def _():
            pltpu.make_async_copy(
                yb.at[s], y_hbm.at[pl.ds(wid * EPW, B)], ysem.at[s]).wait()

        @pl.loop(0, B, unroll=4)
        def _(i):
            ev = ba[pl.ds(k * (3 * B) + 3 * i, 16)]
            e0 = ev[0]
            e1 = ev[1]
            e2 = ev[2]
            for h in range(HEADS):
                sl = pl.ds(h * DH, DH)
                yb[s, i, sl] = (b1[s, i, sl] + b2[s, i, sl]
                                + e0 * wbuf[0, sl] + e1 * wbuf[1, sl]
                                + e2 * wbuf[2, sl])

        pltpu.make_async_copy(
            yb.at[s], y_hbm.at[pl.ds(wid * EPW + k * B, B)],
            ysem.at[s]).start()

    issue(0, 0)

    @pl.loop(0, NB, step=2)
    def _(k):
        issue(k + 1, 1)
        drain(0)
        compute(k, 0)

        @pl.when(k + 2 < NB)
        def _():
            issue(k + 2, 0)

        drain(1)
        compute(k + 1, 1)

    for s in range(2):
        pltpu.make_async_copy(yb.at[s], y_hbm.at[pl.ds(wid * EPW, B)],
                              ysem.at[s]).wait()


def _sc_edge(a1, a2, eaflat, pk2, w1a):
    f = pl.kernel(
        _sc_edge_body,
        out_type=jax.ShapeDtypeStruct((EPAD, H), F32),
        mesh=plsc.VectorSubcoreMesh(**_MESH),
        compiler_params=_SC_PARAMS_TILED,
        scratch_types=[
            pltpu.VMEM((NB, B), I32),
            pltpu.VMEM((2, B), I32),
            pltpu.VMEM((2, B), I32),
            pltpu.VMEM((2, B, H), F32),
            pltpu.VMEM((2, B, H), F32),
            pltpu.VMEM((_EAW + 16,), F32),
            pltpu.VMEM((2, B, H), F32),
            pltpu.VMEM((3, H), F32),
            pltpu.SemaphoreType.DMA((2,)),
            pltpu.SemaphoreType.DMA((2,)),
        ],
    )
    return f(a1, a2, eaflat, pk2, w1a)


def _blockdiag_attn(a):
    return (a.reshape(HEADS, DH, 1)
            * jnp.eye(HEADS, dtype=a.dtype)[:, None, :]).reshape(H, HEADS)


def kernel(x, edge_attr, params, edge_index):
    p = params
    src = edge_index[0]
    dst = edge_index[1]
    npad = EPAD - E
    srcp = jnp.concatenate([src, jnp.zeros((npad,), I32)])
    dstp = jnp.concatenate([dst, jnp.full((npad,), N, I32)])
    pk2 = (srcp | (dstp << 14)).reshape(-1, B)
    eaflat = jnp.concatenate(
        [edge_attr, jnp.zeros((npad, 3), F32)]).reshape(-1)
    wa = jnp.concatenate(
        [_blockdiag_attn(p['gat_asrc']), _blockdiag_attn(p['gat_adst'])],
        axis=1)

    h0, hh, asd, mx = _k1(x, wa, p)
    zrow = jnp.zeros((NPAD - N, 2 * HEADS), F32)
    ts = jnp.concatenate([asd, zrow], axis=0)
    td = jnp.concatenate(
        [jnp.concatenate([asd[:, HEADS:], asd[:, :HEADS]], axis=1), zrow],
        axis=0)

    den_p, num_p = _sc_gat(ts, td, hh, pk2, mx)
    h1 = _k2(den_p, num_p, asd, mx, hh, h0, p)

    agg1 = _sc_agg(h1, pk2)
    h2 = _k3(agg1, den_p, h1, p['s1_Wl'], p['s1_bl'], p['s1_Wr'],
             p['bn1_g'], p['bn1_b'], H, resid=True)

    agg2 = _sc_agg(h2, pk2)
    node = _k3(agg2, den_p, h2, p['s2_Wl'], p['s2_bl'], p['s2_Wr'],
               p['bn2_g'], p['bn2_b'], OUT, resid=False)

    a1, a2 = _k4a(node, p)
    y = _sc_edge(a1, a2, eaflat, pk2, p['cf_W1'][2 * OUT:])
    t, loc, perf, st5 = _k4h(node, p)
    sup = _k4b(t, st5, p)
    stats = _tc_stats(y)
    carbon = _tc_carbon(y, stats, p)
    return (node, carbon, sup, loc, perf)


# speedup vs baseline: 1.0128x; 1.0128x over previous
import functools
import jax
import jax.numpy as jnp
from jax import lax
from jax.experimental import pallas as pl
from jax.experimental.pallas import tpu as pltpu
from jax.experimental.pallas import tpu_sc as plsc

N = 10000
E = 320000
IN = 128
H = 128
OUT = 64
HEADS = 8
DH = 16

NC = 2
NS = 16
NW = NC * NS
B = 64
EPW = 10112
NB = EPW // B
EPAD = EPW * NW
NPAD = 10016
RPT = NPAD // NS
ZB = 128
RCH = RPT // ZB
REM = RPT % ZB

NBLK = 1000
NNB = N // NBLK

_MESH = dict(core_axis_name="c", subcore_axis_name="s", num_cores=NC,
             num_subcores=NS)
_SC_PARAMS = pltpu.CompilerParams(needs_layout_passes=False,
                                  use_tc_tiling_on_sc=False)
_SC_PARAMS_TILED = _SC_PARAMS

F32 = jnp.float32
I32 = jnp.int32


def _leaky(t):
    return jnp.where(t > 0, t, 0.2 * t)


def _rowspec(d):
    return pl.BlockSpec((NBLK, d), lambda k: (k, 0))


def _fullspec(shape):
    nd = len(shape)
    return pl.BlockSpec(shape, lambda k, _n=nd: (0,) * _n)


def _partspec(d):
    return pl.BlockSpec((2, NBLK, d), lambda k: (0, k, 0))


def _affine(stats_row, g, beta):
    mu = stats_row[0] / N
    var = stats_row[1] / N - mu * mu
    s = g * lax.rsqrt(var + 1e-5)
    return s, beta - mu * s


def _acc_stats(acc_ref, out_ref, y, k):
    @pl.when(k == 0)
    def _():
        acc_ref[...] = jnp.zeros_like(acc_ref)

    acc_ref[...] += jnp.concatenate(
        [jnp.sum(y, axis=0)[None], jnp.sum(y * y, axis=0)[None]], axis=0)

    @pl.when(k == NNB - 1)
    def _():
        out_ref[...] = acc_ref[...]


def _k1a_body(x_ref, w_ref, b_ref, y_ref, s_ref, acc_ref):
    y = x_ref[...] @ w_ref[...] + b_ref[...]
    y_ref[...] = y
    _acc_stats(acc_ref, s_ref, y, pl.program_id(0))


def _k1a(x, p):
    return pl.pallas_call(
        _k1a_body,
        grid=(NNB,),
        in_specs=[_rowspec(IN), _fullspec((IN, H)), _fullspec((H,))],
        out_specs=(_rowspec(H), _fullspec((2, H))),
        out_shape=(jax.ShapeDtypeStruct((N, H), F32),
                   jax.ShapeDtypeStruct((2, H), F32)),
        scratch_shapes=[pltpu.VMEM((2, H), F32)],
    )(x, p['in_W'], p['in_b'])


def _k1b_body(y_ref, s_ref, g_ref, beta_ref, gw_ref, wa_ref, h0_ref, hh_ref,
              asd_ref, mx_ref, acc_ref):
    k = pl.program_id(0)
    s, c = _affine(s_ref[...], g_ref[...], beta_ref[...])
    h0 = jax.nn.relu(y_ref[...] * s + c)
    hh = h0 @ gw_ref[...]
    asd = hh @ wa_ref[...]
    h0_ref[...] = h0
    hh_ref[...] = hh
    asd_ref[...] = asd

    @pl.when(k == 0)
    def _():
        acc_ref[...] = jnp.full_like(acc_ref, -jnp.inf)

    acc_ref[...] = jnp.maximum(acc_ref[...], jnp.max(asd, axis=0)[None])

    @pl.when(k == NNB - 1)
    def _():
        mx = acc_ref[...]
        swapped = jnp.concatenate([mx[:, HEADS:], mx[:, :HEADS]], axis=1)
        mx_ref[...] = jnp.concatenate([mx, swapped], axis=0)


def _k1b(y, stats, wa, p):
    return pl.pallas_call(
        _k1b_body,
        grid=(NNB,),
        in_specs=[_rowspec(H), _fullspec((2, H)), _fullspec((H,)),
                  _fullspec((H,)), _fullspec((H, H)),
                  _fullspec((H, 2 * HEADS))],
        out_specs=(_rowspec(H), _rowspec(H), _rowspec(2 * HEADS),
                   _fullspec((2, 2 * HEADS))),
        out_shape=(jax.ShapeDtypeStruct((N, H), F32),
                   jax.ShapeDtypeStruct((N, H), F32),
                   jax.ShapeDtypeStruct((N, 2 * HEADS), F32),
                   jax.ShapeDtypeStruct((2, 2 * HEADS), F32)),
        scratch_shapes=[pltpu.VMEM((1, 2 * HEADS), F32)],
    )(y, stats, p['in_g'], p['in_beta'], p['gat_W'], wa)


def _k2a_body(denp_ref, nump_ref, asd_ref, mx_ref, hh_ref, gb_ref, gat_ref,
              s_ref, acc_ref):
    a_s = asd_ref[...][:, :HEADS]
    a_d = asd_ref[...][:, HEADS:]
    mx = mx_ref[...]
    m = _leaky(mx[0:1, :HEADS] + mx[0:1, HEADS:])
    vs = jnp.exp(_leaky(a_s + a_d) - m)
    den = (denp_ref[...][0, :, :HEADS] + denp_ref[...][1, :, :HEADS]
           + vs + 1e-16)
    hh = hh_ref[...]
    vrep = jnp.repeat(vs, DH, axis=1)
    drep = jnp.repeat(den, DH, axis=1)
    num = nump_ref[...][0] + nump_ref[...][1] + hh * vrep
    gat = num / drep + gb_ref[...]
    gat_ref[...] = gat
    _acc_stats(acc_ref, s_ref, gat, pl.program_id(0))


def _k2a(den_p, num_p, asd, mx, hh, p):
    return pl.pallas_call(
        _k2a_body,
        grid=(NNB,),
        in_specs=[_partspec(2 * HEADS), _partspec(H), _rowspec(2 * HEADS),
                  _fullspec((2, 2 * HEADS)), _rowspec(H), _fullspec((H,))],
        out_specs=(_rowspec(H), _fullspec((2, H))),
        out_shape=(jax.ShapeDtypeStruct((N, H), F32),
                   jax.ShapeDtypeStruct((2, H), F32)),
        scratch_shapes=[pltpu.VMEM((2, H), F32)],
    )(den_p, num_p, asd, mx, hh, p['gat_b'])


def _bnres_body(y_ref, s_ref, g_ref, beta_ref, h_ref, o_ref):
    s, c = _affine(s_ref[...], g_ref[...], beta_ref[...])
    o_ref[...] = jax.nn.relu(y_ref[...] * s + c) + h_ref[...]


def _bnonly_body(y_ref, s_ref, g_ref, beta_ref, o_ref):
    s, c = _affine(s_ref[...], g_ref[...], beta_ref[...])
    o_ref[...] = y_ref[...] * s + c


def _bnapply(y, stats, g, beta, h, d, resid):
    in_specs = [_rowspec(d), _fullspec((2, d)), _fullspec((d,)),
                _fullspec((d,))]
    args = [y, stats, g, beta]
    if resid:
        in_specs.append(_rowspec(d))
        args.append(h)
    return pl.pallas_call(
        _bnres_body if resid else _bnonly_body,
        grid=(NNB,),
        in_specs=in_specs,
        out_specs=_rowspec(d),
        out_shape=jax.ShapeDtypeStruct((N, d), F32),
    )(*args)


def _k3a_body(aggp_ref, denp_ref, h_ref, wl_ref, bl_ref, wr_ref, y_ref,
              s_ref, acc_ref):
    deg = (denp_ref[...][0, :, HEADS:HEADS + 1]
           + denp_ref[...][1, :, HEADS:HEADS + 1])
    inv = 1.0 / jnp.maximum(deg, 1.0)
    agg = (aggp_ref[...][0] + aggp_ref[...][1]) * inv
    y = agg @ wl_ref[...] + bl_ref[...] + h_ref[...] @ wr_ref[...]
    y_ref[...] = y
    _acc_stats(acc_ref, s_ref, y, pl.program_id(0))


def _k3a(agg_p, den_p, h, wl, bl, wr, dout):
    return pl.pallas_call(
        _k3a_body,
        grid=(NNB,),
        in_specs=[_partspec(H), _partspec(2 * HEADS), _rowspec(H),
                  _fullspec((H, dout)), _fullspec((dout,)),
                  _fullspec((H, dout))],
        out_specs=(_rowspec(dout), _fullspec((2, dout))),
        out_shape=(jax.ShapeDtypeStruct((N, dout), F32),
                   jax.ShapeDtypeStruct((2, dout), F32)),
        scratch_shapes=[pltpu.VMEM((2, dout), F32)],
    )(agg_p, den_p, h, wl, bl, wr)


def _k4a_body(node_ref, w1u_ref, w1v_ref, a1_ref, a2_ref):
    node = node_ref[...]
    a1_ref[...] = node @ w1u_ref[...]
    a2_ref[...] = node @ w1v_ref[...]


def _k4a(node, p):
    return pl.pallas_call(
        _k4a_body,
        grid=(NNB,),
        in_specs=[_rowspec(OUT), _fullspec((OUT, H)), _fullspec((OUT, H))],
        out_specs=(_rowspec(H), _rowspec(H)),
        out_shape=(jax.ShapeDtypeStruct((N, H), F32),
                   jax.ShapeDtypeStruct((N, H), F32)),
    )(node, p['cf_W1'][:OUT], p['cf_W1'][OUT:2 * OUT])


def _k4h_body(node_ref, sw1_ref, sb1_ref, lw_ref, lb_ref, pw_ref, pb_ref,
              t_ref, loc_ref, perf_ref, s_ref, acc_ref):
    node = node_ref[...]
    t = node @ sw1_ref[...] + sb1_ref[...]
    t_ref[...] = t
    loc_ref[...] = node @ lw_ref[...] + lb_ref[...]
    perf_ref[...] = node @ pw_ref[...] + pb_ref[...]
    _acc_stats(acc_ref, s_ref, t, pl.program_id(0))


def _k4h(node, p):
    return pl.pallas_call(
        _k4h_body,
        grid=(NNB,),
        in_specs=[_rowspec(OUT), _fullspec((OUT, H)), _fullspec((H,)),
                  _fullspec((OUT, 3)), _fullspec((3,)), _fullspec((OUT, 1)),
                  _fullspec((1,))],
        out_specs=(_rowspec(H), _rowspec(3), _rowspec(1), _fullspec((2, H))),
        out_shape=(jax.ShapeDtypeStruct((N, H), F32),
                   jax.ShapeDtypeStruct((N, 3), F32),
                   jax.ShapeDtypeStruct((N, 1), F32),
                   jax.ShapeDtypeStruct((2, H), F32)),
        scratch_shapes=[pltpu.VMEM((2, H), F32)],
    )(node, p['sc_W1'], p['sc_b1'], p['loc_W'], p['loc_b'], p['perf_W'],
      p['perf_b'])


def _k4b_body(t_ref, s_ref, g_ref, beta_ref, w2_ref, b2_ref, w3_ref, b3_ref,
              sup_ref):
    s, c = _affine(s_ref[...], g_ref[...], beta_ref[...])
    z = jax.nn.relu(t_ref[...] * s + c)
    z = jax.nn.relu(z @ w2_ref[...] + b2_ref[...])
    sup_ref[...] = z @ w3_ref[...] + b3_ref[...]


def _k4b(t, stats, p):
    return pl.pallas_call(
        _k4b_body,
        grid=(NNB,),
        in_specs=[_rowspec(H), _fullspec((2, H)), _fullspec((H,)),
                  _fullspec((H,)), _fullspec((H, OUT)), _fullspec((OUT,)),
                  _fullspec((OUT, 4)), _fullspec((4,))],
        out_specs=_rowspec(4),
        out_shape=jax.ShapeDtypeStruct((N, 4), F32),
    )(t, stats, p['sc_g'], p['sc_beta'], p['sc_W2'], p['sc_b2'], p['sc_W3'],
      p['sc_b3'])


_YBLK = 512
_NYB = E // _YBLK


def _stats_body(y_ref, s_ref, acc_ref):
    k = pl.program_id(0)

    @pl.when(k == 0)
    def _():
        acc_ref[...] = jnp.zeros_like(acc_ref)

    y = y_ref[...]
    acc_ref[...] += jnp.concatenate(
        [jnp.sum(y, axis=0)[None], jnp.sum(y * y, axis=0)[None]], axis=0)

    @pl.when(k == _NYB - 1)
    def _():
        s_ref[...] = acc_ref[...]


def _tc_stats(y):
    return pl.pallas_call(
        _stats_body,
        grid=(_NYB,),
        in_specs=[pl.BlockSpec((_YBLK, H), lambda k: (k, 0))],
        out_specs=pl.BlockSpec((2, H), lambda k: (0, 0)),
        out_shape=jax.ShapeDtypeStruct((2, H), F32),
        scratch_shapes=[pltpu.VMEM((2, H), F32)],
    )(y)


def _carbon_body(y_ref, s_ref, g_ref, beta_ref, w2_ref, b2_ref, w3_ref,
                 b3_ref, out_ref):
    mu = s_ref[...][0] / E
    var = s_ref[...][1] / E - mu * mu
    sc = g_ref[...] * lax.rsqrt(var + 1e-5)
    cc = beta_ref[...] - mu * sc
    z = jax.nn.relu(y_ref[...] * sc + cc)
    z = jax.nn.relu(z @ w2_ref[...] + b2_ref[...])
    out_ref[...] = z @ w3_ref[...] + b3_ref[...]


def _tc_carbon(y, stats, p):
    return pl.pallas_call(
        _carbon_body,
        grid=(_NYB,),
        in_specs=[
            pl.BlockSpec((_YBLK, H), lambda k: (k, 0)),
            pl.BlockSpec((2, H), lambda k: (0, 0)),
            pl.BlockSpec((H,), lambda k: (0,)),
            pl.BlockSpec((H,), lambda k: (0,)),
            pl.BlockSpec((H, OUT), lambda k: (0, 0)),
            pl.BlockSpec((OUT,), lambda k: (0,)),
            pl.BlockSpec((OUT, 1), lambda k: (0, 0)),
            pl.BlockSpec((1,), lambda k: (0,)),
        ],
        out_specs=pl.BlockSpec((_YBLK, 1), lambda k: (k, 0)),
        out_shape=jax.ShapeDtypeStruct((E, 1), F32),
    )(y, stats, p['cf_g'], p['cf_beta'], p['cf_W2'], p['cf_b2'], p['cf_W3'],
      p['cf_b3'])


def _zero_rows(buf, rows):
    cols = buf.shape[1]
    zero = jnp.zeros((16,), F32)

    @pl.loop(0, rows)
    def _(r):
        @pl.loop(0, cols, step=16)
        def _(c0):
            buf[r, pl.ds(c0, 16)] = zero


def _zero_spmem(zb, dst_s, r0):
    rows = zb.shape[0]
    n_full = RPT // rows
    rem = RPT % rows
    for j in range(n_full):
        pltpu.sync_copy(zb, dst_s.at[pl.ds(r0 + j * rows, rows)])
    if rem:
        pltpu.sync_copy(zb.at[pl.ds(0, rem)],
                        dst_s.at[pl.ds(r0 + n_full * rows, rem)])


def _unpack_idx(pk, ixs_b, ixd_b, k, s):
    @pl.loop(0, B, step=16)
    def _(c):
        p = pk[k, pl.ds(c, 16)]
        ixs_b[s, pl.ds(c, 16)] = p & 0x3FFF
        ixd_b[s, pl.ds(c, 16)] = lax.shift_right_logical(p, 14)


def _sc_gat_body(ts_hbm, td_hbm, hh_hbm, pk_hbm, m_hbm,
                 den_hbm, num_hbm,
                 pk, ixs, ixd, bs, bd, bh, vb, nb_, mbuf, den_s, num_s, gsem,
                 esem):
    cid = lax.axis_index("c")
    sid = lax.axis_index("s")
    wid = sid * NC + cid
    r0 = sid * RPT

    _zero_rows(nb_, B)
    _zero_rows(vb, B)
    _zero_spmem(nb_, num_s, r0)
    _zero_spmem(vb, den_s, r0)
    pltpu.sync_copy(m_hbm, mbuf)
    pltpu.sync_copy(pk_hbm.at[pl.ds(wid * NB, NB)], pk)
    plsc.subcore_barrier()

    lane = lax.iota(I32, 16)
    is_head = lane < HEADS
    is_deg = lane == HEADS
    mr = mbuf[0] + mbuf[1]
    mvec = jnp.maximum(mr, 0.2 * mr)

    def issue(k, s):
        _unpack_idx(pk, ixs, ixd, k, s)
        pltpu.make_async_copy(ts_hbm.at[ixs.at[s]], bs.at[s],
                              gsem.at[s]).start()
        pltpu.make_async_copy(td_hbm.at[ixd.at[s]], bd.at[s],
                              gsem.at[s]).start()
        pltpu.make_async_copy(hh_hbm.at[ixs.at[s]], bh.at[s],
                              gsem.at[s]).start()

    def drain(s):
        pltpu.make_async_copy(ts_hbm.at[ixs.at[s]], bs.at[s],
                              gsem.at[s]).wait()
        pltpu.make_async_copy(td_hbm.at[ixd.at[s]], bd.at[s],
                              gsem.at[s]).wait()
        pltpu.make_async_copy(hh_hbm.at[ixs.at[s]], bh.at[s],
                              gsem.at[s]).wait()

    def compute(s):
        @pl.loop(0, B, unroll=4)
        def _(i):
            t = bs[s, i] + bd[s, i]
            v = jnp.exp(jnp.maximum(t, 0.2 * t) - mvec)
            v = jnp.where(is_head, v,
                          jnp.where(is_deg, jnp.ones((16,), F32),
                                    jnp.zeros((16,), F32)))
            vb[i] = v
            for h in range(HEADS):
                vh = plsc.load_gather(
                    vb, [jnp.full((16,), i, I32), jnp.full((16,), h, I32)])
                nb_[i, pl.ds(h * DH, DH)] = bh[s, i, pl.ds(h * DH, DH)] * vh

        pltpu.sync_copy(vb, den_s.at[ixd.at[s]], add=True)
        pltpu.sync_copy(nb_, num_s.at[ixd.at[s]], add=True)

    issue(0, 0)

    @pl.loop(0, NB, step=2)
    def _(k):
        issue(k + 1, 1)
        drain(0)
        compute(0)

        @pl.when(k + 2 < NB)
        def _():
            issue(k + 2, 0)

        drain(1)
        compute(1)

    plsc.subcore_barrier()
    pltpu.async_copy(den_s.at[pl.ds(r0, RPT)],
                     den_hbm.at[cid, pl.ds(r0, RPT)], esem).wait()
    pltpu.async_copy(num_s.at[pl.ds(r0, RPT)],
                     num_hbm.at[cid, pl.ds(r0, RPT)], esem).wait()


def _sc_gat(ts, td, hh, pk2, m):
    f = pl.kernel(
        _sc_gat_body,
        out_type=(
            jax.ShapeDtypeStruct((NC, NPAD, 16), F32),
            jax.ShapeDtypeStruct((NC, NPAD, H), F32),
        ),
        mesh=plsc.VectorSubcoreMesh(**_MESH),
        compiler_params=_SC_PARAMS,
        scratch_types=[
            pltpu.VMEM((NB, B), I32),
            pltpu.VMEM((2, B), I32),
            pltpu.VMEM((2, B), I32),
            pltpu.VMEM((2, B, 16), F32),
            pltpu.VMEM((2, B, 16), F32),
            pltpu.VMEM((2, B, H), F32),
            pltpu.VMEM((B, 16), F32),
            pltpu.VMEM((B, H), F32),
            pltpu.VMEM((2, 16), F32),
            pltpu.VMEM_SHARED((NPAD, 16), F32),
            pltpu.VMEM_SHARED((NPAD, H), F32),
            pltpu.SemaphoreType.DMA((2,)),
            pltpu.SemaphoreType.DMA,
        ],
    )
    return f(ts, td, hh, pk2, m)


def _sc_agg_body(h_hbm, pk_hbm, agg_hbm, pk, ixs, ixd, bh, acc_s,
                 gsem, esem):
    cid = lax.axis_index("c")
    sid = lax.axis_index("s")
    wid = sid * NC + cid
    r0 = sid * RPT

    _zero_rows(bh.at[0], B)
    _zero_spmem(bh.at[0], acc_s, r0)
    pltpu.sync_copy(pk_hbm.at[pl.ds(wid * NB, NB)], pk)
    plsc.subcore_barrier()

    def issue(k, s):
        _unpack_idx(pk, ixs, ixd, k, s)
        pltpu.make_async_copy(h_hbm.at[ixs.at[s]], bh.at[s],
                              gsem.at[s]).start()

    def flush(s):
        pltpu.make_async_copy(h_hbm.at[ixs.at[s]], bh.at[s],
                              gsem.at[s]).wait()
        pltpu.sync_copy(bh.at[s], acc_s.at[ixd.at[s]], add=True)

    issue(0, 0)

    @pl.loop(0, NB, step=2)
    def _(k):
        issue(k + 1, 1)
        flush(0)

        @pl.when(k + 2 < NB)
        def _():
            issue(k + 2, 0)

        flush(1)

    plsc.subcore_barrier()
    pltpu.async_copy(acc_s.at[pl.ds(r0, RPT)],
                     agg_hbm.at[cid, pl.ds(r0, RPT)], esem).wait()


def _sc_agg(h, pk2):
    f = pl.kernel(
        _sc_agg_body,
        out_type=jax.ShapeDtypeStruct((NC, NPAD, H), F32),
        mesh=plsc.VectorSubcoreMesh(**_MESH),
        compiler_params=_SC_PARAMS_TILED,
        scratch_types=[
            pltpu.VMEM((NB, B), I32),
            pltpu.VMEM((2, B), I32),
            pltpu.VMEM((2, B), I32),
            pltpu.VMEM((2, B, H), F32),
            pltpu.VMEM_SHARED((NPAD, H), F32),
            pltpu.SemaphoreType.DMA((2,)),
            pltpu.SemaphoreType.DMA,
        ],
    )
    return f(h, pk2)


_EAW = 3 * EPW


def _sc_edge_body(a1_hbm, a2_hbm, ea_hbm, pk_hbm, w1a_hbm, y_hbm,
                  pk, ixs, ixd, b1, b2, ba, yb, wbuf, gsem, ysem):
    cid = lax.axis_index("c")
    sid = lax.axis_index("s")
    wid = sid * NC + cid

    pltpu.sync_copy(w1a_hbm, wbuf)
    pltpu.sync_copy(pk_hbm.at[pl.ds(wid * NB, NB)], pk)
    pltpu.sync_copy(ea_hbm.at[pl.ds(wid * _EAW, _EAW)],
                    ba.at[pl.ds(0, _EAW)])

    def issue(k, s):
        _unpack_idx(pk, ixs, ixd, k, s)
        pltpu.make_async_copy(a1_hbm.at[ixs.at[s]], b1.at[s],
                              gsem.at[s]).start()
        pltpu.make_async_copy(a2_hbm.at[ixd.at[s]], b2.at[s],
                              gsem.at[s]).start()

    def drain(s):
        pltpu.make_async_copy(a1_hbm.at[ixs.at[s]], b1.at[s],
                              gsem.at[s]).wait()
        pltpu.make_async_copy(a2_hbm.at[ixd.at[s]], b2.at[s],
                              gsem.at[s]).wait()

    def compute(k, s):
        @pl.when(k >= 2)
        def _():
            pltpu.make_async_copy(
                yb.at[s], y_hbm.at[pl.ds(wid * EPW, B)], ysem.at[s]).wait()

        @pl.loop(0, B, unroll=4)
        def _(i):
            ev = ba[pl.ds(k * (3 * B) + 3 * i, 16)]
            e0 = ev[0]
            e1 = ev[1]
            e2 = ev[2]
            for h in range(HEADS):
                sl = pl.ds(h * DH, DH)
                yb[s, i, sl] = (b1[s, i, sl] + b2[s, i, sl]
                                + e0 * wbuf[0, sl] + e1 * wbuf[1, sl]
                                + e2 * wbuf[2, sl])

        pltpu.make_async_copy(
            yb.at[s], y_hbm.at[pl.ds(wid * EPW + k * B, B)],
            ysem.at[s]).start()

    issue(0, 0)

    @pl.loop(0, NB, step=2)
    def _(k):
        issue(k + 1, 1)
        drain(0)
        compute(k, 0)

        @pl.when(k + 2 < NB)
        def _():
            issue(k + 2, 0)

        drain(1)
        compute(k + 1, 1)

    for s in range(2):
        pltpu.make_async_copy(yb.at[s], y_hbm.at[pl.ds(wid * EPW, B)],
                              ysem.at[s]).wait()


def _sc_edge(a1, a2, eaflat, pk2, w1a):
    f = pl.kernel(
        _sc_edge_body,
        out_type=jax.ShapeDtypeStruct((EPAD, H), F32),
        mesh=plsc.VectorSubcoreMesh(**_MESH),
        compiler_params=_SC_PARAMS_TILED,
        scratch_types=[
            pltpu.VMEM((NB, B), I32),
            pltpu.VMEM((2, B), I32),
            pltpu.VMEM((2, B), I32),
            pltpu.VMEM((2, B, H), F32),
            pltpu.VMEM((2, B, H), F32),
            pltpu.VMEM((_EAW + 16,), F32),
            pltpu.VMEM((2, B, H), F32),
            pltpu.VMEM((3, H), F32),
            pltpu.SemaphoreType.DMA((2,)),
            pltpu.SemaphoreType.DMA((2,)),
        ],
    )
    return f(a1, a2, eaflat, pk2, w1a)


def _blockdiag_attn(a):
    return (a.reshape(HEADS, DH, 1)
            * jnp.eye(HEADS, dtype=a.dtype)[:, None, :]).reshape(H, HEADS)


def kernel(x, edge_attr, params, edge_index):
    p = params
    src = edge_index[0]
    dst = edge_index[1]
    npad = EPAD - E
    srcp = jnp.concatenate([src, jnp.zeros((npad,), I32)])
    dstp = jnp.concatenate([dst, jnp.full((npad,), N, I32)])
    pk2 = (srcp | (dstp << 14)).reshape(-1, B)
    eaflat = jnp.concatenate(
        [edge_attr, jnp.zeros((npad, 3), F32)]).reshape(-1)
    wa = jnp.concatenate(
        [_blockdiag_attn(p['gat_asrc']), _blockdiag_attn(p['gat_adst'])],
        axis=1)

    y1, st1 = _k1a(x, p)
    h0, hh, asd, mx = _k1b(y1, st1, wa, p)
    zrow = jnp.zeros((NPAD - N, 2 * HEADS), F32)
    ts = jnp.concatenate([asd, zrow], axis=0)
    td = jnp.concatenate(
        [jnp.concatenate([asd[:, HEADS:], asd[:, :HEADS]], axis=1), zrow],
        axis=0)

    den_p, num_p = _sc_gat(ts, td, hh, pk2, mx)
    gat, st2 = _k2a(den_p, num_p, asd, mx, hh, p)
    h1 = _bnapply(gat, st2, p['bn0_g'], p['bn0_b'], h0, H, resid=True)

    agg1 = _sc_agg(h1, pk2)
    y3, st3 = _k3a(agg1, den_p, h1, p['s1_Wl'], p['s1_bl'], p['s1_Wr'], H)
    h2 = _bnapply(y3, st3, p['bn1_g'], p['bn1_b'], h1, H, resid=True)

    agg2 = _sc_agg(h2, pk2)
    y4, st4 = _k3a(agg2, den_p, h2, p['s2_Wl'], p['s2_bl'], p['s2_Wr'], OUT)
    node = _bnapply(y4, st4, p['bn2_g'], p['bn2_b'], None, OUT, resid=False)

    a1, a2 = _k4a(node, p)
    y = _sc_edge(a1, a2, eaflat, pk2, p['cf_W1'][2 * OUT:])
    t, loc, perf, st5 = _k4h(node, p)
    sup = _k4b(t, st5, p)
    stats = _tc_stats(y)
    carbon = _tc_carbon(y, stats, p)
    return (node, carbon, sup, loc, perf)


# speedup vs baseline: 1.0131x; 1.0003x over previous
import jax
import jax.numpy as jnp
from jax import lax
from jax.experimental import pallas as pl
from jax.experimental.pallas import tpu as pltpu
from jax.experimental.pallas import tpu_sc as plsc

N = 10000
E = 320000
IN = 128
H = 128
OUT = 64
HEADS = 8
DH = 16

NC = 2
NS = 16
NW = NC * NS
B = 64
EPW = 10112
NB = EPW // B
EPAD = EPW * NW
NPAD = 10016
RPT = NPAD // NS
ZB = 128
RCH = RPT // ZB
REM = RPT % ZB

NBLK = 1000
NNB = N // NBLK

_MESH = dict(core_axis_name="c", subcore_axis_name="s", num_cores=NC,
             num_subcores=NS)
_SC_PARAMS = pltpu.CompilerParams(needs_layout_passes=False,
                                  use_tc_tiling_on_sc=False)
_SC_PARAMS_TILED = _SC_PARAMS

F32 = jnp.float32
I32 = jnp.int32


def _leaky(t):
    return jnp.where(t > 0, t, 0.2 * t)


def _rowspec(d):
    return pl.BlockSpec((NBLK, d), lambda k: (k, 0))


def _fullspec(shape):
    nd = len(shape)
    return pl.BlockSpec(shape, lambda k, _n=nd: (0,) * _n)


def _partspec(d):
    return pl.BlockSpec((2, NBLK, d), lambda k: (0, k, 0))


def _affine(stats_row, g, beta):
    mu = stats_row[0] / N
    var = stats_row[1] / N - mu * mu
    s = g * lax.rsqrt(var + 1e-5)
    return s, beta - mu * s


def _acc_stats(acc_ref, out_ref, y, k):
    @pl.when(k == 0)
    def _():
        acc_ref[...] = jnp.zeros_like(acc_ref)

    acc_ref[...] += jnp.concatenate(
        [jnp.sum(y, axis=0)[None], jnp.sum(y * y, axis=0)[None]], axis=0)

    @pl.when(k == NNB - 1)
    def _():
        out_ref[...] = acc_ref[...]


def _k1a_body(x_ref, w_ref, b_ref, y_ref, s_ref, acc_ref):
    y = x_ref[...] @ w_ref[...] + b_ref[...]
    y_ref[...] = y
    _acc_stats(acc_ref, s_ref, y, pl.program_id(0))


def _k1a(x, p):
    return pl.pallas_call(
        _k1a_body,
        grid=(NNB,),
        in_specs=[_rowspec(IN), _fullspec((IN, H)), _fullspec((H,))],
        out_specs=(_rowspec(H), _fullspec((2, H))),
        out_shape=(jax.ShapeDtypeStruct((N, H), F32),
                   jax.ShapeDtypeStruct((2, H), F32)),
        scratch_shapes=[pltpu.VMEM((2, H), F32)],
    )(x, p['in_W'], p['in_b'])


def _k1b_body(y_ref, s_ref, g_ref, beta_ref, gw_ref, wa_ref, h0_ref, hh_ref,
              asd_ref, mx_ref, acc_ref):
    k = pl.program_id(0)
    s, c = _affine(s_ref[...], g_ref[...], beta_ref[...])
    h0 = jax.nn.relu(y_ref[...] * s + c)
    hh = h0 @ gw_ref[...]
    asd = hh @ wa_ref[...]
    h0_ref[...] = h0
    hh_ref[...] = hh
    asd_ref[...] = asd

    @pl.when(k == 0)
    def _():
        acc_ref[...] = jnp.full_like(acc_ref, -jnp.inf)

    acc_ref[...] = jnp.maximum(acc_ref[...], jnp.max(asd, axis=0)[None])

    @pl.when(k == NNB - 1)
    def _():
        mx = acc_ref[...]
        swapped = jnp.concatenate([mx[:, HEADS:], mx[:, :HEADS]], axis=1)
        mx_ref[...] = jnp.concatenate([mx, swapped], axis=0)


def _k1b(y, stats, wa, p):
    return pl.pallas_call(
        _k1b_body,
        grid=(NNB,),
        in_specs=[_rowspec(H), _fullspec((2, H)), _fullspec((H,)),
                  _fullspec((H,)), _fullspec((H, H)),
                  _fullspec((H, 2 * HEADS))],
        out_specs=(_rowspec(H), _rowspec(H), _rowspec(2 * HEADS),
                   _fullspec((2, 2 * HEADS))),
        out_shape=(jax.ShapeDtypeStruct((N, H), F32),
                   jax.ShapeDtypeStruct((N, H), F32),
                   jax.ShapeDtypeStruct((N, 2 * HEADS), F32),
                   jax.ShapeDtypeStruct((2, 2 * HEADS), F32)),
        scratch_shapes=[pltpu.VMEM((1, 2 * HEADS), F32)],
    )(y, stats, p['in_g'], p['in_beta'], p['gat_W'], wa)


def _k2a_body(denp_ref, nump_ref, asd_ref, mx_ref, hh_ref, gb_ref, gat_ref,
              s_ref, acc_ref):
    a_s = asd_ref[...][:, :HEADS]
    a_d = asd_ref[...][:, HEADS:]
    mx = mx_ref[...]
    m = _leaky(mx[0:1, :HEADS] + mx[0:1, HEADS:])
    vs = jnp.exp(_leaky(a_s + a_d) - m)
    den = (denp_ref[...][0, :, :HEADS] + denp_ref[...][1, :, :HEADS]
           + vs + 1e-16)
    hh = hh_ref[...]
    vrep = jnp.repeat(vs, DH, axis=1)
    drep = jnp.repeat(den, DH, axis=1)
    num = nump_ref[...][0] + nump_ref[...][1] + hh * vrep
    gat = num / drep + gb_ref[...]
    gat_ref[...] = gat
    _acc_stats(acc_ref, s_ref, gat, pl.program_id(0))


def _k2a(den_p, num_p, asd, mx, hh, p):
    return pl.pallas_call(
        _k2a_body,
        grid=(NNB,),
        in_specs=[_partspec(2 * HEADS), _partspec(H), _rowspec(2 * HEADS),
                  _fullspec((2, 2 * HEADS)), _rowspec(H), _fullspec((H,))],
        out_specs=(_rowspec(H), _fullspec((2, H))),
        out_shape=(jax.ShapeDtypeStruct((N, H), F32),
                   jax.ShapeDtypeStruct((2, H), F32)),
        scratch_shapes=[pltpu.VMEM((2, H), F32)],
    )(den_p, num_p, asd, mx, hh, p['gat_b'])


def _bnres_body(y_ref, s_ref, g_ref, beta_ref, h_ref, o_ref):
    s, c = _affine(s_ref[...], g_ref[...], beta_ref[...])
    o_ref[...] = jax.nn.relu(y_ref[...] * s + c) + h_ref[...]


def _bnonly_body(y_ref, s_ref, g_ref, beta_ref, o_ref):
    s, c = _affine(s_ref[...], g_ref[...], beta_ref[...])
    o_ref[...] = y_ref[...] * s + c


def _bnapply(y, stats, g, beta, h, d, resid):
    in_specs = [_rowspec(d), _fullspec((2, d)), _fullspec((d,)),
                _fullspec((d,))]
    args = [y, stats, g, beta]
    if resid:
        in_specs.append(_rowspec(d))
        args.append(h)
    return pl.pallas_call(
        _bnres_body if resid else _bnonly_body,
        grid=(NNB,),
        in_specs=in_specs,
        out_specs=_rowspec(d),
        out_shape=jax.ShapeDtypeStruct((N, d), F32),
    )(*args)


def _k3a_body(aggp_ref, denp_ref, h_ref, wl_ref, bl_ref, wr_ref, y_ref,
              s_ref, acc_ref):
    deg = (denp_ref[...][0, :, HEADS:HEADS + 1]
           + denp_ref[...][1, :, HEADS:HEADS + 1])
    inv = 1.0 / jnp.maximum(deg, 1.0)
    agg = (aggp_ref[...][0] + aggp_ref[...][1]) * inv
    y = agg @ wl_ref[...] + bl_ref[...] + h_ref[...] @ wr_ref[...]
    y_ref[...] = y
    _acc_stats(acc_ref, s_ref, y, pl.program_id(0))


def _k3a(agg_p, den_p, h, wl, bl, wr, dout):
    return pl.pallas_call(
        _k3a_body,
        grid=(NNB,),
        in_specs=[_partspec(H), _partspec(2 * HEADS), _rowspec(H),
                  _fullspec((H, dout)), _fullspec((dout,)),
                  _fullspec((H, dout))],
        out_specs=(_rowspec(dout), _fullspec((2, dout))),
        out_shape=(jax.ShapeDtypeStruct((N, dout), F32),
                   jax.ShapeDtypeStruct((2, dout), F32)),
        scratch_shapes=[pltpu.VMEM((2, dout), F32)],
    )(agg_p, den_p, h, wl, bl, wr)


def _k4a_body(node_ref, w1u_ref, w1v_ref, a1_ref, a2_ref):
    node = node_ref[...]
    a1_ref[...] = node @ w1u_ref[...]
    a2_ref[...] = node @ w1v_ref[...]


def _k4a(node, p):
    return pl.pallas_call(
        _k4a_body,
        grid=(NNB,),
        in_specs=[_rowspec(OUT), _fullspec((OUT, H)), _fullspec((OUT, H))],
        out_specs=(_rowspec(H), _rowspec(H)),
        out_shape=(jax.ShapeDtypeStruct((N, H), F32),
                   jax.ShapeDtypeStruct((N, H), F32)),
    )(node, p['cf_W1'][:OUT], p['cf_W1'][OUT:2 * OUT])


def _k4h_body(node_ref, sw1_ref, sb1_ref, lw_ref, lb_ref, pw_ref, pb_ref,
              t_ref, loc_ref, perf_ref, s_ref, acc_ref):
    node = node_ref[...]
    t = node @ sw1_ref[...] + sb1_ref[...]
    t_ref[...] = t
    loc_ref[...] = node @ lw_ref[...] + lb_ref[...]
    perf_ref[...] = node @ pw_ref[...] + pb_ref[...]
    _acc_stats(acc_ref, s_ref, t, pl.program_id(0))


def _k4h(node, p):
    return pl.pallas_call(
        _k4h_body,
        grid=(NNB,),
        in_specs=[_rowspec(OUT), _fullspec((OUT, H)), _fullspec((H,)),
                  _fullspec((OUT, 3)), _fullspec((3,)), _fullspec((OUT, 1)),
                  _fullspec((1,))],
        out_specs=(_rowspec(H), _rowspec(3), _rowspec(1), _fullspec((2, H))),
        out_shape=(jax.ShapeDtypeStruct((N, H), F32),
                   jax.ShapeDtypeStruct((N, 3), F32),
                   jax.ShapeDtypeStruct((N, 1), F32),
                   jax.ShapeDtypeStruct((2, H), F32)),
        scratch_shapes=[pltpu.VMEM((2, H), F32)],
    )(node, p['sc_W1'], p['sc_b1'], p['loc_W'], p['loc_b'], p['perf_W'],
      p['perf_b'])


def _k4b_body(t_ref, s_ref, g_ref, beta_ref, w2_ref, b2_ref, w3_ref, b3_ref,
              sup_ref):
    s, c = _affine(s_ref[...], g_ref[...], beta_ref[...])
    z = jax.nn.relu(t_ref[...] * s + c)
    z = jax.nn.relu(z @ w2_ref[...] + b2_ref[...])
    sup_ref[...] = z @ w3_ref[...] + b3_ref[...]


def _k4b(t, stats, p):
    return pl.pallas_call(
        _k4b_body,
        grid=(NNB,),
        in_specs=[_rowspec(H), _fullspec((2, H)), _fullspec((H,)),
                  _fullspec((H,)), _fullspec((H, OUT)), _fullspec((OUT,)),
                  _fullspec((OUT, 4)), _fullspec((4,))],
        out_specs=_rowspec(4),
        out_shape=jax.ShapeDtypeStruct((N, 4), F32),
    )(t, stats, p['sc_g'], p['sc_beta'], p['sc_W2'], p['sc_b2'], p['sc_W3'],
      p['sc_b3'])


_YBLK = 512
_NYB = E // _YBLK


def _stats_body(y_ref, s_ref, acc_ref):
    k = pl.program_id(0)

    @pl.when(k == 0)
    def _():
        acc_ref[...] = jnp.zeros_like(acc_ref)

    y = y_ref[...]
    acc_ref[...] += jnp.concatenate(
        [jnp.sum(y, axis=0)[None], jnp.sum(y * y, axis=0)[None]], axis=0)

    @pl.when(k == _NYB - 1)
    def _():
        s_ref[...] = acc_ref[...]


def _tc_stats(y):
    return pl.pallas_call(
        _stats_body,
        grid=(_NYB,),
        in_specs=[pl.BlockSpec((_YBLK, H), lambda k: (k, 0))],
        out_specs=pl.BlockSpec((2, H), lambda k: (0, 0)),
        out_shape=jax.ShapeDtypeStruct((2, H), F32),
        scratch_shapes=[pltpu.VMEM((2, H), F32)],
    )(y)


def _carbon_body(y_ref, s_ref, g_ref, beta_ref, w2_ref, b2_ref, w3_ref,
                 b3_ref, out_ref):
    mu = s_ref[...][0] / E
    var = s_ref[...][1] / E - mu * mu
    sc = g_ref[...] * lax.rsqrt(var + 1e-5)
    cc = beta_ref[...] - mu * sc
    z = jax.nn.relu(y_ref[...] * sc + cc)
    z = jax.nn.relu(z @ w2_ref[...] + b2_ref[...])
    out_ref[...] = z @ w3_ref[...] + b3_ref[...]


def _tc_carbon(y, stats, p):
    return pl.pallas_call(
        _carbon_body,
        grid=(_NYB,),
        in_specs=[
            pl.BlockSpec((_YBLK, H), lambda k: (k, 0)),
            pl.BlockSpec((2, H), lambda k: (0, 0)),
            pl.BlockSpec((H,), lambda k: (0,)),
            pl.BlockSpec((H,), lambda k: (0,)),
            pl.BlockSpec((H, OUT), lambda k: (0, 0)),
            pl.BlockSpec((OUT,), lambda k: (0,)),
            pl.BlockSpec((OUT, 1), lambda k: (0, 0)),
            pl.BlockSpec((1,), lambda k: (0,)),
        ],
        out_specs=pl.BlockSpec((_YBLK, 1), lambda k: (k, 0)),
        out_shape=jax.ShapeDtypeStruct((E, 1), F32),
    )(y, stats, p['cf_g'], p['cf_beta'], p['cf_W2'], p['cf_b2'], p['cf_W3'],
      p['cf_b3'])


def _zero_rows(buf, rows):
    cols = buf.shape[1]
    zero = jnp.zeros((16,), F32)

    @pl.loop(0, rows)
    def _(r):
        @pl.loop(0, cols, step=16)
        def _(c0):
            buf[r, pl.ds(c0, 16)] = zero


def _zero_spmem(zb, dst_s, r0):
    rows = zb.shape[0]
    n_full = RPT // rows
    rem = RPT % rows
    for j in range(n_full):
        pltpu.sync_copy(zb, dst_s.at[pl.ds(r0 + j * rows, rows)])
    if rem:
        pltpu.sync_copy(zb.at[pl.ds(0, rem)],
                        dst_s.at[pl.ds(r0 + n_full * rows, rem)])


def _unpack_idx(pk, ixs_b, ixd_b, k, s):
    @pl.loop(0, B, step=16)
    def _(c):
        p = pk[k, pl.ds(c, 16)]
        ixs_b[s, pl.ds(c, 16)] = p & 0x3FFF
        ixd_b[s, pl.ds(c, 16)] = lax.shift_right_logical(p, 14)


def _sc_gat_body(ts_hbm, td_hbm, hh_hbm, pk_hbm, m_hbm,
                 den_hbm, num_hbm,
                 pk, ixs, ixd, bs, bd, bh, vb, nb_, mbuf, den_s, num_s, gsem,
                 esem):
    cid = lax.axis_index("c")
    sid = lax.axis_index("s")
    wid = sid * NC + cid
    r0 = sid * RPT

    _zero_rows(nb_, B)
    _zero_rows(vb, B)
    _zero_spmem(nb_, num_s, r0)
    _zero_spmem(vb, den_s, r0)
    pltpu.sync_copy(m_hbm, mbuf)
    pltpu.sync_copy(pk_hbm.at[pl.ds(wid * NB, NB)], pk)
    plsc.subcore_barrier()

    lane = lax.iota(I32, 16)
    is_head = lane < HEADS
    is_deg = lane == HEADS
    mr = mbuf[0] + mbuf[1]
    mvec = jnp.maximum(mr, 0.2 * mr)

    def issue(k, s):
        _unpack_idx(pk, ixs, ixd, k, s)
        pltpu.make_async_copy(ts_hbm.at[ixs.at[s]], bs.at[s],
                              gsem.at[s]).start()
        pltpu.make_async_copy(td_hbm.at[ixd.at[s]], bd.at[s],
                              gsem.at[s]).start()
        pltpu.make_async_copy(hh_hbm.at[ixs.at[s]], bh.at[s],
                              gsem.at[s]).start()

    def drain(s):
        pltpu.make_async_copy(ts_hbm.at[ixs.at[s]], bs.at[s],
                              gsem.at[s]).wait()
        pltpu.make_async_copy(td_hbm.at[ixd.at[s]], bd.at[s],
                              gsem.at[s]).wait()
        pltpu.make_async_copy(hh_hbm.at[ixs.at[s]], bh.at[s],
                              gsem.at[s]).wait()

    def compute(s):
        @pl.loop(0, B, unroll=4)
        def _(i):
            t = bs[s, i] + bd[s, i]
            v = jnp.exp(jnp.maximum(t, 0.2 * t) - mvec)
            v = jnp.where(is_head, v,
                          jnp.where(is_deg, jnp.ones((16,), F32),
                                    jnp.zeros((16,), F32)))
            vb[i] = v
            for h in range(HEADS):
                vh = plsc.load_gather(
                    vb, [jnp.full((16,), i, I32), jnp.full((16,), h, I32)])
                nb_[i, pl.ds(h * DH, DH)] = bh[s, i, pl.ds(h * DH, DH)] * vh

        pltpu.sync_copy(vb, den_s.at[ixd.at[s]], add=True)
        pltpu.sync_copy(nb_, num_s.at[ixd.at[s]], add=True)

    issue(0, 0)

    @pl.loop(0, NB, step=2)
    def _(k):
        issue(k + 1, 1)
        drain(0)
        compute(0)

        @pl.when(k + 2 < NB)
        def _():
            issue(k + 2, 0)

        drain(1)
        compute(1)

    plsc.subcore_barrier()
    pltpu.async_copy(den_s.at[pl.ds(r0, RPT)],
                     den_hbm.at[cid, pl.ds(r0, RPT)], esem).wait()
    pltpu.async_copy(num_s.at[pl.ds(r0, RPT)],
                     num_hbm.at[cid, pl.ds(r0, RPT)], esem).wait()


def _sc_gat(ts, td, hh, pk2, m):
    f = pl.kernel(
        _sc_gat_body,
        out_type=(
            jax.ShapeDtypeStruct((NC, NPAD, 16), F32),
            jax.ShapeDtypeStruct((NC, NPAD, H), F32),
        ),
        mesh=plsc.VectorSubcoreMesh(**_MESH),
        compiler_params=_SC_PARAMS,
        scratch_types=[
            pltpu.VMEM((NB, B), I32),
            pltpu.VMEM((2, B), I32),
            pltpu.VMEM((2, B), I32),
            pltpu.VMEM((2, B, 16), F32),
            pltpu.VMEM((2, B, 16), F32),
            pltpu.VMEM((2, B, H), F32),
            pltpu.VMEM((B, 16), F32),
            pltpu.VMEM((B, H), F32),
            pltpu.VMEM((2, 16), F32),
            pltpu.VMEM_SHARED((NPAD, 16), F32),
            pltpu.VMEM_SHARED((NPAD, H), F32),
            pltpu.SemaphoreType.DMA((2,)),
            pltpu.SemaphoreType.DMA,
        ],
    )
    return f(ts, td, hh, pk2, m)


def _sc_agg_body(h_hbm, pk_hbm, agg_hbm, pk, ixs, ixd, bh, acc_s,
                 gsem, esem):
    cid = lax.axis_index("c")
    sid = lax.axis_index("s")
    wid = sid * NC + cid
    r0 = sid * RPT

    _zero_rows(bh.at[0], B)
    _zero_spmem(bh.at[0], acc_s, r0)
    pltpu.sync_copy(pk_hbm.at[pl.ds(wid * NB, NB)], pk)
    plsc.subcore_barrier()

    def issue(k, s):
        _unpack_idx(pk, ixs, ixd, k, s)
        pltpu.make_async_copy(h_hbm.at[ixs.at[s]], bh.at[s],
                              gsem.at[s]).start()

    def flush(s):
        pltpu.make_async_copy(h_hbm.at[ixs.at[s]], bh.at[s],
                              gsem.at[s]).wait()
        pltpu.sync_copy(bh.at[s], acc_s.at[ixd.at[s]], add=True)

    issue(0, 0)

    @pl.loop(0, NB, step=2)
    def _(k):
        issue(k + 1, 1)
        flush(0)

        @pl.when(k + 2 < NB)
        def _():
            issue(k + 2, 0)

        flush(1)

    plsc.subcore_barrier()
    pltpu.async_copy(acc_s.at[pl.ds(r0, RPT)],
                     agg_hbm.at[cid, pl.ds(r0, RPT)], esem).wait()


def _sc_agg(h, pk2):
    f = pl.kernel(
        _sc_agg_body,
        out_type=jax.ShapeDtypeStruct((NC, NPAD, H), F32),
        mesh=plsc.VectorSubcoreMesh(**_MESH),
        compiler_params=_SC_PARAMS_TILED,
        scratch_types=[
            pltpu.VMEM((NB, B), I32),
            pltpu.VMEM((2, B), I32),
            pltpu.VMEM((2, B), I32),
            pltpu.VMEM((2, B, H), F32),
            pltpu.VMEM_SHARED((NPAD, H), F32),
            pltpu.SemaphoreType.DMA((2,)),
            pltpu.SemaphoreType.DMA,
        ],
    )
    return f(h, pk2)


_EAW = 3 * EPW


def _sc_edge_body(a1_hbm, a2_hbm, ea_hbm, pk_hbm, w1a_hbm, y_hbm,
                  pk, ixs, ixd, b1, b2, ba, yb, wbuf, gsem, ysem):
    cid = lax.axis_index("c")
    sid = lax.axis_index("s")
    wid = sid * NC + cid

    pltpu.sync_copy(w1a_hbm, wbuf)
    pltpu.sync_copy(pk_hbm.at[pl.ds(wid * NB, NB)], pk)
    pltpu.sync_copy(ea_hbm.at[pl.ds(wid * _EAW, _EAW)],
                    ba.at[pl.ds(0, _EAW)])

    def issue(k, s):
        _unpack_idx(pk, ixs, ixd, k, s)
        pltpu.make_async_copy(a1_hbm.at[ixs.at[s]], b1.at[s],
                              gsem.at[s]).start()
        pltpu.make_async_copy(a2_hbm.at[ixd.at[s]], b2.at[s],
                              gsem.at[s]).start()

    def drain(s):
        pltpu.make_async_copy(a1_hbm.at[ixs.at[s]], b1.at[s],
                              gsem.at[s]).wait()
        pltpu.make_async_copy(a2_hbm.at[ixd.at[s]], b2.at[s],
                              gsem.at[s]).wait()

    def compute(k, s):
        @pl.when(k >= 2)
        def _():
            pltpu.make_async_copy(
                yb.at[s], y_hbm.at[pl.ds(wid * EPW, B)], ysem.at[s]).wait()

        @pl.loop(0, B, unroll=4)
        def _(i):
            ev = ba[pl.ds(k * (3 * B) + 3 * i, 16)]
            e0 = ev[0]
            e1 = ev[1]
            e2 = ev[2]
            for h in range(HEADS):
                sl = pl.ds(h * DH, DH)
                yb[s, i, sl] = (b1[s, i, sl] + b2[s, i, sl]
                                + e0 * wbuf[0, sl] + e1 * wbuf[1, sl]
                                + e2 * wbuf[2, sl])

        pltpu.make_async_copy(
            yb.at[s], y_hbm.at[pl.ds(wid * EPW + k * B, B)],
            ysem.at[s]).start()

    issue(0, 0)

    @pl.loop(0, NB, step=2)
    def _(k):
        issue(k + 1, 1)
        drain(0)
        compute(k, 0)

        @pl.when(k + 2 < NB)
        def _():
            issue(k + 2, 0)

        drain(1)
        compute(k + 1, 1)

    for s in range(2):
        pltpu.make_async_copy(yb.at[s], y_hbm.at[pl.ds(wid * EPW, B)],
                              ysem.at[s]).wait()


def _sc_edge(a1, a2, eaflat, pk2, w1a):
    f = pl.kernel(
        _sc_edge_body,
        out_type=jax.ShapeDtypeStruct((EPAD, H), F32),
        mesh=plsc.VectorSubcoreMesh(**_MESH),
        compiler_params=_SC_PARAMS_TILED,
        scratch_types=[
            pltpu.VMEM((NB, B), I32),
            pltpu.VMEM((2, B), I32),
            pltpu.VMEM((2, B), I32),
            pltpu.VMEM((2, B, H), F32),
            pltpu.VMEM((2, B, H), F32),
            pltpu.VMEM((_EAW + 16,), F32),
            pltpu.VMEM((2, B, H), F32),
            pltpu.VMEM((3, H), F32),
            pltpu.SemaphoreType.DMA((2,)),
            pltpu.SemaphoreType.DMA((2,)),
        ],
    )
    return f(a1, a2, eaflat, pk2, w1a)


def _blockdiag_attn(a):
    return (a.reshape(HEADS, DH, 1)
            * jnp.eye(HEADS, dtype=a.dtype)[:, None, :]).reshape(H, HEADS)


def kernel(x, edge_attr, params, edge_index):
    p = params
    src = edge_index[0]
    dst = edge_index[1]
    npad = EPAD - E
    srcp = jnp.concatenate([src, jnp.zeros((npad,), I32)])
    dstp = jnp.concatenate([dst, jnp.full((npad,), N, I32)])
    pk2 = (srcp | (dstp << 14)).reshape(-1, B)
    eaflat = jnp.concatenate(
        [edge_attr, jnp.zeros((npad, 3), F32)]).reshape(-1)
    wa = jnp.concatenate(
        [_blockdiag_attn(p['gat_asrc']), _blockdiag_attn(p['gat_adst'])],
        axis=1)

    y1, st1 = _k1a(x, p)
    h0, hh, asd, mx = _k1b(y1, st1, wa, p)
    zrow = jnp.zeros((NPAD - N, 2 * HEADS), F32)
    ts = jnp.concatenate([asd, zrow], axis=0)
    td = jnp.concatenate(
        [jnp.concatenate([asd[:, HEADS:], asd[:, :HEADS]], axis=1), zrow],
        axis=0)

    den_p, num_p = _sc_gat(ts, td, hh, pk2, mx)
    gat, st2 = _k2a(den_p, num_p, asd, mx, hh, p)
    h1 = _bnapply(gat, st2, p['bn0_g'], p['bn0_b'], h0, H, resid=True)

    agg1 = _sc_agg(h1, pk2)
    y3, st3 = _k3a(agg1, den_p, h1, p['s1_Wl'], p['s1_bl'], p['s1_Wr'], H)
    h2 = _bnapply(y3, st3, p['bn1_g'], p['bn1_b'], h1, H, resid=True)

    agg2 = _sc_agg(h2, pk2)
    y4, st4 = _k3a(agg2, den_p, h2, p['s2_Wl'], p['s2_bl'], p['s2_Wr'], OUT)
    node = _bnapply(y4, st4, p['bn2_g'], p['bn2_b'], None, OUT, resid=False)

    a1, a2 = _k4a(node, p)
    y = _sc_edge(a1, a2, eaflat, pk2, p['cf_W1'][2 * OUT:])
    t, loc, perf, st5 = _k4h(node, p)
    sup = _k4b(t, st5, p)
    stats = _tc_stats(y)
    carbon = _tc_carbon(y, stats, p)
    return (node, carbon, sup, loc, perf)
